# baseline (speedup 1.0000x reference)
import functools

import jax
import jax.numpy as jnp
from jax import lax
from jax.experimental import pallas as pl
from jax.experimental.pallas import tpu as pltpu
from jax.experimental.pallas import tpu_sc as plsc

N = 32768
D = 1024
BLOCK = 2048

NC = 2
NS = 16
L = 16
NW = NC * NS
ROWS_PER_W = N // NW
RCH = 8
NCHUNK = ROWS_PER_W // RCH


def _pre_kernel(x_ref, wg_ref, bg_ref, pre_ref):
    xb = x_ref[...]
    z = jnp.dot(xb, wg_ref[...], preferred_element_type=jnp.float32)
    z = z + bg_ref[...]
    s = jax.nn.sigmoid(z)
    m = (z > 0.0).astype(jnp.float32)
    sm = s * m
    w_pre = sm[:, 0:1] + sm[:, 1:2]
    pre_ref[...] = xb * w_pre


def _tc_pre(x, wg_p, bg_p):
    grid = (N // BLOCK,)
    return pl.pallas_call(
        _pre_kernel,
        grid=grid,
        in_specs=[
            pl.BlockSpec((BLOCK, D), lambda i: (i, 0)),
            pl.BlockSpec((D, 128), lambda i: (0, 0)),
            pl.BlockSpec((1, 128), lambda i: (0, 0)),
        ],
        out_specs=pl.BlockSpec((BLOCK, D), lambda i: (i, 0)),
        out_shape=jax.ShapeDtypeStruct((N, D), jnp.float32),
    )(x, wg_p, bg_p)


def _sc_post_body(x_hbm, wgt_hbm, bgb_hbm, out_hbm, xbuf, obuf, wgbuf, bgbuf):
    wid = lax.axis_index("s") * NC + lax.axis_index("c")
    base0 = wid * ROWS_PER_W
    pltpu.sync_copy(wgt_hbm, wgbuf)
    pltpu.sync_copy(bgb_hbm, bgbuf)
    bg0 = bgbuf[0, :]
    bg1 = bgbuf[1, :]

    def wg_round(i, c):
        off = i * L
        for rr in range(2):
            v = wgbuf[rr, pl.ds(off, L)]
            t = v * 65537.0
            wgbuf[rr, pl.ds(off, L)] = t - (t - v)
        return c

    lax.fori_loop(0, D // L, wg_round, 0)

    def chunk_body(g, carry):
        base = base0 + g * RCH
        pltpu.sync_copy(x_hbm.at[pl.ds(base, RCH)], xbuf)

        def dot_body(i, accs):
            off = i * L
            w0 = wgbuf[0, pl.ds(off, L)]
            w1 = wgbuf[1, pl.ds(off, L)]
            new = []
            for r in range(RCH):
                xa = xbuf[r, pl.ds(off, L)]
                t = xa * 65537.0
                xa = t - (t - xa)
                new.append(accs[2 * r] + xa * w0)
                new.append(accs[2 * r + 1] + xa * w1)
            return tuple(new)

        accs = lax.fori_loop(
            0, D // L, dot_body,
            tuple(jnp.zeros((L,), jnp.float32) for _ in range(2 * RCH)),
        )

        wposts = []
        for r in range(RCH):
            z0 = jnp.full((L,), jnp.sum(accs[2 * r])) + bg0
            z1 = jnp.full((L,), jnp.sum(accs[2 * r + 1])) + bg1
            s0 = 1.0 / (1.0 + jnp.exp(-z0))
            s1 = 1.0 / (1.0 + jnp.exp(-z1))
            m0 = (z0 > 0.0).astype(jnp.float32)
            m1 = (z1 > 0.0).astype(jnp.float32)
            w_pre = s0 * m0 + s1 * m1
            wposts.append(w_pre * (m0 + m1))

        def scale_body(i, c):
            off = i * L
            for r in range(RCH):
                obuf[r, pl.ds(off, L)] = xbuf[r, pl.ds(off, L)] * wposts[r]
            return c

        lax.fori_loop(0, D // L, scale_body, 0)
        pltpu.sync_copy(obuf, out_hbm.at[pl.ds(base, RCH)])
        return carry

    lax.fori_loop(0, NCHUNK, chunk_body, 0)


def _build_sc_post(interpret=False):
    return pl.kernel(
        _sc_post_body,
        mesh=plsc.VectorSubcoreMesh(core_axis_name="c", subcore_axis_name="s"),
        out_type=jax.ShapeDtypeStruct((N, D), jnp.float32),
        compiler_params=pltpu.CompilerParams(needs_layout_passes=False),
        scratch_types=[
            pltpu.VMEM((RCH, D), jnp.float32),
            pltpu.VMEM((RCH, D), jnp.float32),
            pltpu.VMEM((2, D), jnp.float32),
            pltpu.VMEM((2, L), jnp.float32),
        ],
        interpret=interpret,
    )


_sc_post = _build_sc_post()


@jax.jit
def kernel(x, Wg, bg):
    wg_p = jnp.zeros((D, 128), dtype=jnp.float32).at[:, :2].set(Wg)
    bg_p = jnp.zeros((1, 128), dtype=jnp.float32).at[0, :2].set(bg)
    pre = _tc_pre(x, wg_p, bg_p)
    wgt = Wg.T.reshape(2, D)
    bgb = jnp.broadcast_to(bg.reshape(2, 1), (2, L))
    post = _sc_post(x, wgt, bgb)
    return (pre, post)


# speedup vs baseline: 3.5028x; 3.5028x over previous
import jax
import jax.numpy as jnp
from jax.experimental import pallas as pl

N = 32768
D = 1024
BLOCK = 2048


def _route_kernel(x_ref, wg_ref, bg_ref, pre_ref, post_ref):
    xb = x_ref[...]
    z = jnp.dot(xb, wg_ref[...], preferred_element_type=jnp.float32)
    z = z + bg_ref[...]
    s = jax.nn.sigmoid(z)
    m = (z > 0.0).astype(jnp.float32)
    sm = s * m
    w_pre = sm[:, 0:1] + sm[:, 1:2]
    w_post = w_pre * (m[:, 0:1] + m[:, 1:2])
    pre_ref[...] = xb * w_pre
    post_ref[...] = xb * w_post


@jax.jit
def kernel(x, Wg, bg):
    wg_p = jnp.zeros((D, 128), dtype=jnp.float32).at[:, :2].set(Wg)
    bg_p = jnp.zeros((1, 128), dtype=jnp.float32).at[0, :2].set(bg)
    grid = (N // BLOCK,)
    pre, post = pl.pallas_call(
        _route_kernel,
        grid=grid,
        in_specs=[
            pl.BlockSpec((BLOCK, D), lambda i: (i, 0)),
            pl.BlockSpec((D, 128), lambda i: (0, 0)),
            pl.BlockSpec((1, 128), lambda i: (0, 0)),
        ],
        out_specs=[
            pl.BlockSpec((BLOCK, D), lambda i: (i, 0)),
            pl.BlockSpec((BLOCK, D), lambda i: (i, 0)),
        ],
        out_shape=[
            jax.ShapeDtypeStruct((N, D), jnp.float32),
            jax.ShapeDtypeStruct((N, D), jnp.float32),
        ],
    )(x, wg_p, bg_p)
    return (pre, post)
